# baseline (speedup 1.0000x reference)
import jax
import jax.numpy as jnp
from jax import lax
from jax.experimental import pallas as pl
from jax.experimental.pallas import tpu as pltpu
from jax.experimental.pallas import tpu_sc as plsc

N = 10000
E = 160000
D = 256
DH = 128
DG = 8
NSUB = 16
EPT = E // NSUB
CH = 80
NCH = EPT // CH
RPT = 632
NP = NSUB * RPT
BN = RPT

NBUF = 3
NIDX = 6


def _sc_agg_body(x_hbm, edge_hbm, ones_hbm, agg_hbm, deg_hbm,
                 agg_sh, deg_sh, ones_v, *rest):
    bufs = rest[0:NBUF]
    rvs = rest[NBUF:NBUF + NIDX]
    svs = rest[NBUF + NIDX:NBUF + 2 * NIDX]
    dvs = rest[NBUF + 2 * NIDX:NBUF + 3 * NIDX]
    semg = rest[NBUF + 3 * NIDX:2 * NBUF + 3 * NIDX]
    sems = rest[2 * NBUF + 3 * NIDX:3 * NBUF + 3 * NIDX]
    semd = rest[3 * NBUF + 3 * NIDX:4 * NBUF + 3 * NIDX]
    semi = rest[4 * NBUF + 3 * NIDX:4 * NBUF + 4 * NIDX]
    c = lax.axis_index("c")
    s = lax.axis_index("s")

    pltpu.sync_copy(ones_hbm, ones_v)

    def zfill(i, carry):
        for j in range(DH // 16):
            bufs[0][i, pl.ds(j * 16, 16)] = jnp.zeros((16,), jnp.float32)
        return carry

    lax.fori_loop(0, 8, zfill, 0)

    def zcopy(k, carry):
        pltpu.sync_copy(bufs[0].at[pl.ds(0, 8)],
                        agg_sh.at[pl.ds(s * RPT + k * 8, 8)])
        pltpu.sync_copy(bufs[0].at[pl.ds(0, 8), pl.ds(0, DG)],
                        deg_sh.at[pl.ds(s * RPT + k * 8, 8)])
        return carry

    lax.fori_loop(0, RPT // 8, zcopy, 0)
    plsc.subcore_barrier()

    ebase = s * EPT

    def prep(q, k):
        pltpu.async_copy(edge_hbm.at[pl.ds(ebase + k * CH, CH)],
                         rvs[q], semi[q])
        pltpu.async_copy(edge_hbm.at[pl.ds(E + ebase + k * CH, CH)],
                         dvs[q], semi[q])

    def idxwait(q):
        pltpu.make_async_copy(edge_hbm.at[pl.ds(0, CH)], rvs[q],
                              semi[q]).wait()
        pltpu.make_async_copy(edge_hbm.at[pl.ds(0, CH)], dvs[q],
                              semi[q]).wait()
        for j in range(CH // 16):
            v = rvs[q][pl.ds(j * 16, 16)]
            svs[q][pl.ds(j * 16, 16)] = v + v + c

    def gstart(b, q):
        pltpu.async_copy(x_hbm.at[svs[q]], bufs[b], semg[b])

    def gwait(b, q):
        pltpu.make_async_copy(x_hbm.at[svs[q]], bufs[b], semg[b]).wait()

    def sstart(b, q):
        pltpu.async_copy(bufs[b], agg_sh.at[dvs[q]], sems[b], add=True)
        pltpu.async_copy(ones_v, deg_sh.at[dvs[q]], semd[b], add=True)

    def swait(b, q):
        pltpu.make_async_copy(bufs[b], agg_sh.at[dvs[q]], sems[b]).wait()
        pltpu.make_async_copy(ones_v, deg_sh.at[dvs[q]], semd[b]).wait()

    def step(k, kk):
        b, q = kk % NBUF, kk % NIDX
        swait((kk - 1) % NBUF, (kk - 1) % NIDX)
        prep((kk + 5) % NIDX, k + 5)
        q2, b2 = (kk + 2) % NIDX, (kk + 2) % NBUF
        idxwait(q2)
        gstart(b2, q2)
        gwait(b, q)
        sstart(b, q)

    for k in range(5):
        prep(k % NIDX, k)
    idxwait(0)
    gstart(0, 0)
    idxwait(1)
    gstart(1, 1)
    prep(5, 5)
    idxwait(2)
    gstart(2, 2)
    gwait(0, 0)
    sstart(0, 0)

    NU = (NCH - 11) // 6

    def six(g, carry):
        k0 = 6 * g + 1
        for j in range(6):
            step(k0 + j, 1 + j)
        return carry

    lax.fori_loop(0, NU, six, 0)

    for k in range(6 * NU + 1, NCH - 5):
        step(k, k)

    for k in range(NCH - 5, NCH):
        b, q = k % NBUF, k % NIDX
        swait((k - 1) % NBUF, (k - 1) % NIDX)
        if k + 2 <= NCH - 1:
            q2, b2 = (k + 2) % NIDX, (k + 2) % NBUF
            idxwait(q2)
            gstart(b2, q2)
        gwait(b, q)
        sstart(b, q)
    swait((NCH - 1) % NBUF, (NCH - 1) % NIDX)
    plsc.subcore_barrier()

    pltpu.sync_copy(agg_sh.at[pl.ds(s * RPT, RPT)],
                    agg_hbm.at[pl.ds(c * NP + s * RPT, RPT)])
    pltpu.sync_copy(deg_sh.at[pl.ds(s * RPT, RPT)],
                    deg_hbm.at[pl.ds(c * NP + s * RPT, RPT)])


def _sc_aggregate(x2, edge2, ones8):
    mesh = plsc.VectorSubcoreMesh(core_axis_name="c", subcore_axis_name="s")
    scratch = (
        [pltpu.VMEM_SHARED((NP, DH), jnp.float32),
         pltpu.VMEM_SHARED((NP, DG), jnp.float32),
         pltpu.VMEM((CH, DG), jnp.float32)]
        + [pltpu.VMEM((CH, DH), jnp.float32)] * NBUF
        + [pltpu.VMEM((CH,), jnp.int32)] * NIDX
        + [pltpu.VMEM((CH,), jnp.int32)] * NIDX
        + [pltpu.VMEM((CH,), jnp.int32)] * NIDX
        + [pltpu.SemaphoreType.DMA] * (3 * NBUF + NIDX)
    )
    return pl.kernel(
        _sc_agg_body,
        out_type=(jax.ShapeDtypeStruct((2 * NP, DH), jnp.float32),
                  jax.ShapeDtypeStruct((2 * NP, DG), jnp.float32)),
        mesh=mesh,
        scratch_types=scratch,
        compiler_params=pltpu.CompilerParams(use_tc_tiling_on_sc=False),
    )(x2, edge2, ones8)


def _tc_xwr_body(x_ref, wr_ref, bl_ref, o_ref):
    o_ref[...] = (jnp.dot(x_ref[...], wr_ref[...],
                          preferred_element_type=jnp.float32) + bl_ref[...])


def _tc_xwr(x, W_r, b_l):
    return pl.pallas_call(
        _tc_xwr_body,
        grid=(N // BN + 1,),
        in_specs=[
            pl.BlockSpec((BN, D), lambda i: (i, 0)),
            pl.BlockSpec((D, D), lambda i: (0, 0)),
            pl.BlockSpec((1, D), lambda i: (0, 0)),
        ],
        out_specs=pl.BlockSpec((BN, D), lambda i: (i, 0)),
        out_shape=jax.ShapeDtypeStruct((N, D), jnp.float32),
    )(x, W_r, b_l)


def _tc_body(a0_ref, a1_ref, deg_ref, r_ref, x_ref, wl_ref, g_ref, b_ref,
             o_ref):
    inv = 1.0 / jnp.maximum(deg_ref[:, 0:1], 1.0)
    m0 = a0_ref[...] * inv
    m1 = a1_ref[...] * inv
    h = (jnp.dot(m0, wl_ref[:128, :], preferred_element_type=jnp.float32)
         + jnp.dot(m1, wl_ref[128:, :], preferred_element_type=jnp.float32)
         + r_ref[...])
    mu = jnp.mean(h, axis=-1, keepdims=True)
    dlt = h - mu
    var = jnp.mean(dlt * dlt, axis=-1, keepdims=True)
    hn = dlt * lax.rsqrt(var + 1e-5) * g_ref[...] + b_ref[...]
    ge = 0.5 * hn * (1.0 + lax.erf(hn * 0.7071067811865476))
    o_ref[...] = ge + x_ref[...]


def _tc_finish(agg, deg, r, x, W_l, gamma, beta):
    nb = NP // BN
    return pl.pallas_call(
        _tc_body,
        grid=(N // BN + 1,),
        in_specs=[
            pl.BlockSpec((BN, DH), lambda i: (i, 0)),
            pl.BlockSpec((BN, DH), lambda i: (nb + i, 0)),
            pl.BlockSpec((BN, DG), lambda i: (i, 0)),
            pl.BlockSpec((BN, D), lambda i: (i, 0)),
            pl.BlockSpec((BN, D), lambda i: (i, 0)),
            pl.BlockSpec((D, D), lambda i: (0, 0)),
            pl.BlockSpec((1, D), lambda i: (0, 0)),
            pl.BlockSpec((1, D), lambda i: (0, 0)),
        ],
        out_specs=pl.BlockSpec((BN, D), lambda i: (i, 0)),
        out_shape=jax.ShapeDtypeStruct((N, D), jnp.float32),
    )(agg, agg, deg, r, x, W_l, gamma, beta)


def kernel(x, edge_index, W_l, b_l, W_r, gamma, beta):
    x2 = x.reshape(2 * N, DH)
    edge2 = edge_index.reshape(2 * E).astype(jnp.int32)
    ones8 = jnp.ones((CH, DG), jnp.float32)
    r = _tc_xwr(x, W_r, b_l.reshape(1, D))
    agg, deg = _sc_aggregate(x2, edge2, ones8)
    return _tc_finish(agg, deg, r, x, W_l,
                      gamma.reshape(1, D), beta.reshape(1, D))


# speedup vs baseline: 1.0017x; 1.0017x over previous
import jax
import jax.numpy as jnp
from jax import lax
from jax.experimental import pallas as pl
from jax.experimental.pallas import tpu as pltpu
from jax.experimental.pallas import tpu_sc as plsc

N = 10000
E = 160000
D = 256
DH = 128
DG = 8
NSUB = 16
EPT = E // NSUB
CH = 80
NCH = EPT // CH
RPT = 632
NP = NSUB * RPT
BN = RPT

NBUF = 3
NIDX = 6


def _sc_agg_body(x_hbm, edge_hbm, ones_hbm, agg_hbm, deg_hbm,
                 agg_sh, deg_sh, ones_v, *rest):
    bufs = rest[0:NBUF]
    rvs = rest[NBUF:NBUF + NIDX]
    svs = rest[NBUF + NIDX:NBUF + 2 * NIDX]
    dvs = rest[NBUF + 2 * NIDX:NBUF + 3 * NIDX]
    semg = rest[NBUF + 3 * NIDX:2 * NBUF + 3 * NIDX]
    sems = rest[2 * NBUF + 3 * NIDX:3 * NBUF + 3 * NIDX]
    semd = rest[3 * NBUF + 3 * NIDX:4 * NBUF + 3 * NIDX]
    semi = rest[4 * NBUF + 3 * NIDX:4 * NBUF + 4 * NIDX]
    c = lax.axis_index("c")
    s = lax.axis_index("s")

    pltpu.sync_copy(ones_hbm, ones_v)

    def zfill(i, carry):
        for j in range(DH // 16):
            bufs[0][i, pl.ds(j * 16, 16)] = jnp.zeros((16,), jnp.float32)
        return carry

    lax.fori_loop(0, 8, zfill, 0)

    def zcopy(k, carry):
        pltpu.sync_copy(bufs[0].at[pl.ds(0, 8)],
                        agg_sh.at[pl.ds(s * RPT + k * 8, 8)])
        pltpu.sync_copy(bufs[0].at[pl.ds(0, 8), pl.ds(0, DG)],
                        deg_sh.at[pl.ds(s * RPT + k * 8, 8)])
        return carry

    lax.fori_loop(0, RPT // 8, zcopy, 0)
    plsc.subcore_barrier()

    ebase = s * EPT

    def prep(q, k):
        pltpu.async_copy(edge_hbm.at[0, pl.ds(ebase + k * CH, CH)],
                         rvs[q], semi[q])
        pltpu.async_copy(edge_hbm.at[1, pl.ds(ebase + k * CH, CH)],
                         dvs[q], semi[q])

    def idxwait(q):
        pltpu.make_async_copy(edge_hbm.at[0, pl.ds(0, CH)], rvs[q],
                              semi[q]).wait()
        pltpu.make_async_copy(edge_hbm.at[1, pl.ds(0, CH)], dvs[q],
                              semi[q]).wait()
        for j in range(CH // 16):
            v = rvs[q][pl.ds(j * 16, 16)]
            svs[q][pl.ds(j * 16, 16)] = v + v + c

    def gstart(b, q):
        pltpu.async_copy(x_hbm.at[svs[q]], bufs[b], semg[b])

    def gwait(b, q):
        pltpu.make_async_copy(x_hbm.at[svs[q]], bufs[b], semg[b]).wait()

    def sstart(b, q):
        pltpu.async_copy(bufs[b], agg_sh.at[dvs[q]], sems[b], add=True)
        pltpu.async_copy(ones_v, deg_sh.at[dvs[q]], semd[b], add=True)

    def swait(b, q):
        pltpu.make_async_copy(bufs[b], agg_sh.at[dvs[q]], sems[b]).wait()
        pltpu.make_async_copy(ones_v, deg_sh.at[dvs[q]], semd[b]).wait()

    def step(k, kk):
        b, q = kk % NBUF, kk % NIDX
        swait((kk - 1) % NBUF, (kk - 1) % NIDX)
        prep((kk + 5) % NIDX, k + 5)
        q2, b2 = (kk + 2) % NIDX, (kk + 2) % NBUF
        idxwait(q2)
        gstart(b2, q2)
        gwait(b, q)
        sstart(b, q)

    for k in range(5):
        prep(k % NIDX, k)
    idxwait(0)
    gstart(0, 0)
    idxwait(1)
    gstart(1, 1)
    prep(5, 5)
    idxwait(2)
    gstart(2, 2)
    gwait(0, 0)
    sstart(0, 0)

    NU = (NCH - 11) // 6

    def six(g, carry):
        k0 = 6 * g + 1
        for j in range(6):
            step(k0 + j, 1 + j)
        return carry

    lax.fori_loop(0, NU, six, 0)

    for k in range(6 * NU + 1, NCH - 5):
        step(k, k)

    for k in range(NCH - 5, NCH):
        b, q = k % NBUF, k % NIDX
        swait((k - 1) % NBUF, (k - 1) % NIDX)
        if k + 2 <= NCH - 1:
            q2, b2 = (k + 2) % NIDX, (k + 2) % NBUF
            idxwait(q2)
            gstart(b2, q2)
        gwait(b, q)
        sstart(b, q)
    swait((NCH - 1) % NBUF, (NCH - 1) % NIDX)
    plsc.subcore_barrier()

    pltpu.sync_copy(agg_sh.at[pl.ds(s * RPT, RPT)],
                    agg_hbm.at[pl.ds(c * NP + s * RPT, RPT)])
    pltpu.sync_copy(deg_sh.at[pl.ds(s * RPT, RPT)],
                    deg_hbm.at[pl.ds(c * NP + s * RPT, RPT)])


def _sc_aggregate(x2, edge2, ones8):
    mesh = plsc.VectorSubcoreMesh(core_axis_name="c", subcore_axis_name="s")
    scratch = (
        [pltpu.VMEM_SHARED((NP, DH), jnp.float32),
         pltpu.VMEM_SHARED((NP, DG), jnp.float32),
         pltpu.VMEM((CH, DG), jnp.float32)]
        + [pltpu.VMEM((CH, DH), jnp.float32)] * NBUF
        + [pltpu.VMEM((CH,), jnp.int32)] * NIDX
        + [pltpu.VMEM((CH,), jnp.int32)] * NIDX
        + [pltpu.VMEM((CH,), jnp.int32)] * NIDX
        + [pltpu.SemaphoreType.DMA] * (3 * NBUF + NIDX)
    )
    return pl.kernel(
        _sc_agg_body,
        out_type=(jax.ShapeDtypeStruct((2 * NP, DH), jnp.float32),
                  jax.ShapeDtypeStruct((2 * NP, DG), jnp.float32)),
        mesh=mesh,
        scratch_types=scratch,
        compiler_params=pltpu.CompilerParams(use_tc_tiling_on_sc=False),
    )(x2, edge2, ones8)


def _tc_xwr_body(x_ref, wr_ref, bl_ref, o_ref):
    o_ref[...] = (jnp.dot(x_ref[...], wr_ref[...],
                          preferred_element_type=jnp.float32) + bl_ref[...])


def _tc_xwr(x, W_r, b_l):
    return pl.pallas_call(
        _tc_xwr_body,
        grid=(N // BN + 1,),
        in_specs=[
            pl.BlockSpec((BN, D), lambda i: (i, 0)),
            pl.BlockSpec((D, D), lambda i: (0, 0)),
            pl.BlockSpec((1, D), lambda i: (0, 0)),
        ],
        out_specs=pl.BlockSpec((BN, D), lambda i: (i, 0)),
        out_shape=jax.ShapeDtypeStruct((N, D), jnp.float32),
    )(x, W_r, b_l)


def _tc_body(a0_ref, a1_ref, deg_ref, r_ref, x_ref, wl_ref, g_ref, b_ref,
             o_ref):
    inv = 1.0 / jnp.maximum(deg_ref[:, 0:1], 1.0)
    m0 = a0_ref[...] * inv
    m1 = a1_ref[...] * inv
    h = (jnp.dot(m0, wl_ref[:128, :], preferred_element_type=jnp.float32)
         + jnp.dot(m1, wl_ref[128:, :], preferred_element_type=jnp.float32)
         + r_ref[...])
    mu = jnp.mean(h, axis=-1, keepdims=True)
    dlt = h - mu
    var = jnp.mean(dlt * dlt, axis=-1, keepdims=True)
    hn = dlt * lax.rsqrt(var + 1e-5) * g_ref[...] + b_ref[...]
    ge = 0.5 * hn * (1.0 + lax.erf(hn * 0.7071067811865476))
    o_ref[...] = ge + x_ref[...]


def _tc_finish(agg, deg, r, x, W_l, gamma, beta):
    nb = NP // BN
    return pl.pallas_call(
        _tc_body,
        grid=(N // BN + 1,),
        in_specs=[
            pl.BlockSpec((BN, DH), lambda i: (i, 0)),
            pl.BlockSpec((BN, DH), lambda i: (nb + i, 0)),
            pl.BlockSpec((BN, DG), lambda i: (i, 0)),
            pl.BlockSpec((BN, D), lambda i: (i, 0)),
            pl.BlockSpec((BN, D), lambda i: (i, 0)),
            pl.BlockSpec((D, D), lambda i: (0, 0)),
            pl.BlockSpec((1, D), lambda i: (0, 0)),
            pl.BlockSpec((1, D), lambda i: (0, 0)),
        ],
        out_specs=pl.BlockSpec((BN, D), lambda i: (i, 0)),
        out_shape=jax.ShapeDtypeStruct((N, D), jnp.float32),
    )(agg, agg, deg, r, x, W_l, gamma, beta)


def kernel(x, edge_index, W_l, b_l, W_r, gamma, beta):
    x2 = x.reshape(2 * N, DH)
    edge2 = edge_index.astype(jnp.int32)
    ones8 = jnp.ones((CH, DG), jnp.float32)
    r = _tc_xwr(x, W_r, b_l.reshape(1, D))
    agg, deg = _sc_aggregate(x2, edge2, ones8)
    return _tc_finish(agg, deg, r, x, W_l,
                      gamma.reshape(1, D), beta.reshape(1, D))


# speedup vs baseline: 1.0495x; 1.0477x over previous
import jax
import jax.numpy as jnp
from jax import lax
from jax.experimental import pallas as pl
from jax.experimental.pallas import tpu as pltpu
from jax.experimental.pallas import tpu_sc as plsc

N = 10000
E = 160000
D = 256
DH = 128
DG = 8
NSUB = 16
EPT = E // NSUB
CH = 80
NCH = EPT // CH
RPT = 632
NP = NSUB * RPT
BN = RPT

NBUF = 3
NIDX = 6


def _sc_agg_body(x_hbm, src_hbm, dst_hbm, ones_hbm, agg_hbm, deg_hbm,
                 agg_sh, deg_sh, ones_v, *rest):
    bufs = rest[0:NBUF]
    rvs = rest[NBUF:NBUF + NIDX]
    svs = rest[NBUF + NIDX:NBUF + 2 * NIDX]
    dvs = rest[NBUF + 2 * NIDX:NBUF + 3 * NIDX]
    semg = rest[NBUF + 3 * NIDX:2 * NBUF + 3 * NIDX]
    sems = rest[2 * NBUF + 3 * NIDX:3 * NBUF + 3 * NIDX]
    semd = rest[3 * NBUF + 3 * NIDX:4 * NBUF + 3 * NIDX]
    semi = rest[4 * NBUF + 3 * NIDX:4 * NBUF + 4 * NIDX]
    c = lax.axis_index("c")
    s = lax.axis_index("s")

    pltpu.sync_copy(ones_hbm, ones_v)

    def zfill(i, carry):
        for j in range(DH // 16):
            bufs[0][i, pl.ds(j * 16, 16)] = jnp.zeros((16,), jnp.float32)
        return carry

    lax.fori_loop(0, CH, zfill, 0)
    nz = RPT // CH
    for k in range(nz):
        pltpu.async_copy(bufs[0], agg_sh.at[pl.ds(s * RPT + k * CH, CH)],
                         semg[0])
    rz = RPT - nz * CH
    pltpu.async_copy(bufs[0].at[pl.ds(0, rz)],
                     agg_sh.at[pl.ds(s * RPT + nz * CH, rz)], semg[0])
    for k in range(8):
        pltpu.async_copy(bufs[0].at[pl.ds(0, RPT // 8), pl.ds(0, DG)],
                         deg_sh.at[pl.ds(s * RPT + k * (RPT // 8), RPT // 8)],
                         semg[1])
    for k in range(nz):
        pltpu.make_async_copy(bufs[0], agg_sh.at[pl.ds(0, CH)],
                              semg[0]).wait()
    pltpu.make_async_copy(bufs[0].at[pl.ds(0, rz)],
                          agg_sh.at[pl.ds(0, rz)], semg[0]).wait()
    for k in range(8):
        pltpu.make_async_copy(bufs[0].at[pl.ds(0, RPT // 8), pl.ds(0, DG)],
                              deg_sh.at[pl.ds(0, RPT // 8)], semg[1]).wait()
    plsc.subcore_barrier()

    ebase = s * EPT

    def prep(q, k):
        pltpu.async_copy(src_hbm.at[pl.ds(ebase + k * CH, CH)],
                         rvs[q], semi[q])
        pltpu.async_copy(dst_hbm.at[pl.ds(ebase + k * CH, CH)],
                         dvs[q], semi[q])

    def idxwait(q):
        pltpu.make_async_copy(src_hbm.at[pl.ds(0, CH)], rvs[q],
                              semi[q]).wait()
        pltpu.make_async_copy(dst_hbm.at[pl.ds(0, CH)], dvs[q],
                              semi[q]).wait()
        for j in range(CH // 16):
            v = rvs[q][pl.ds(j * 16, 16)]
            svs[q][pl.ds(j * 16, 16)] = v + v + c

    def gstart(b, q):
        pltpu.async_copy(x_hbm.at[svs[q]], bufs[b], semg[b])

    def gwait(b, q):
        pltpu.make_async_copy(x_hbm.at[svs[q]], bufs[b], semg[b]).wait()

    def sstart(b, q):
        pltpu.async_copy(bufs[b], agg_sh.at[dvs[q]], sems[b], add=True)
        pltpu.async_copy(ones_v, deg_sh.at[dvs[q]], semd[b], add=True)

    def swait(b, q):
        pltpu.make_async_copy(bufs[b], agg_sh.at[dvs[q]], sems[b]).wait()
        pltpu.make_async_copy(ones_v, deg_sh.at[dvs[q]], semd[b]).wait()

    def step(k, kk):
        b, q = kk % NBUF, kk % NIDX
        swait((kk - 1) % NBUF, (kk - 1) % NIDX)
        prep((kk + 5) % NIDX, k + 5)
        q2, b2 = (kk + 2) % NIDX, (kk + 2) % NBUF
        idxwait(q2)
        gstart(b2, q2)
        gwait(b, q)
        sstart(b, q)

    for k in range(5):
        prep(k % NIDX, k)
    idxwait(0)
    gstart(0, 0)
    idxwait(1)
    gstart(1, 1)
    prep(5, 5)
    idxwait(2)
    gstart(2, 2)
    gwait(0, 0)
    sstart(0, 0)

    NU = (NCH - 11) // 6

    def six(g, carry):
        k0 = 6 * g + 1
        for j in range(6):
            step(k0 + j, 1 + j)
        return carry

    lax.fori_loop(0, NU, six, 0)

    for k in range(6 * NU + 1, NCH - 5):
        step(k, k)

    for k in range(NCH - 5, NCH):
        b, q = k % NBUF, k % NIDX
        swait((k - 1) % NBUF, (k - 1) % NIDX)
        if k + 2 <= NCH - 1:
            q2, b2 = (k + 2) % NIDX, (k + 2) % NBUF
            idxwait(q2)
            gstart(b2, q2)
        gwait(b, q)
        sstart(b, q)
    swait((NCH - 1) % NBUF, (NCH - 1) % NIDX)
    plsc.subcore_barrier()

    pltpu.async_copy(deg_sh.at[pl.ds(s * RPT, RPT)],
                     deg_hbm.at[pl.ds(c * NP + s * RPT, RPT)], semg[1])
    pltpu.async_copy(agg_sh.at[pl.ds(s * RPT, RPT)],
                     agg_hbm.at[pl.ds(c * NP + s * RPT, RPT)], semg[0])
    pltpu.make_async_copy(deg_sh.at[pl.ds(s * RPT, RPT)],
                          deg_hbm.at[pl.ds(0, RPT)], semg[1]).wait()
    pltpu.make_async_copy(agg_sh.at[pl.ds(s * RPT, RPT)],
                          agg_hbm.at[pl.ds(0, RPT)], semg[0]).wait()


def _sc_aggregate(x2, src, dst, ones8):
    mesh = plsc.VectorSubcoreMesh(core_axis_name="c", subcore_axis_name="s")
    scratch = (
        [pltpu.VMEM_SHARED((NP, DH), jnp.float32),
         pltpu.VMEM_SHARED((NP, DG), jnp.float32),
         pltpu.VMEM((CH, DG), jnp.float32)]
        + [pltpu.VMEM((CH, DH), jnp.float32)] * NBUF
        + [pltpu.VMEM((CH,), jnp.int32)] * NIDX
        + [pltpu.VMEM((CH,), jnp.int32)] * NIDX
        + [pltpu.VMEM((CH,), jnp.int32)] * NIDX
        + [pltpu.SemaphoreType.DMA] * (3 * NBUF + NIDX)
    )
    return pl.kernel(
        _sc_agg_body,
        out_type=(jax.ShapeDtypeStruct((2 * NP, DH), jnp.float32),
                  jax.ShapeDtypeStruct((2 * NP, DG), jnp.float32)),
        mesh=mesh,
        scratch_types=scratch,
        compiler_params=pltpu.CompilerParams(use_tc_tiling_on_sc=False),
    )(x2, src, dst, ones8)


def _tc_xwr_body(x_ref, wr_ref, bl_ref, o_ref):
    o_ref[...] = (jnp.dot(x_ref[...], wr_ref[...],
                          preferred_element_type=jnp.float32) + bl_ref[...])


def _tc_xwr(x, W_r, b_l):
    return pl.pallas_call(
        _tc_xwr_body,
        grid=(N // BN + 1,),
        in_specs=[
            pl.BlockSpec((BN, D), lambda i: (i, 0)),
            pl.BlockSpec((D, D), lambda i: (0, 0)),
            pl.BlockSpec((1, D), lambda i: (0, 0)),
        ],
        out_specs=pl.BlockSpec((BN, D), lambda i: (i, 0)),
        out_shape=jax.ShapeDtypeStruct((N, D), jnp.float32),
    )(x, W_r, b_l)


def _tc_body(a0_ref, a1_ref, deg_ref, r_ref, x_ref, wl_ref, g_ref, b_ref,
             o_ref):
    inv = 1.0 / jnp.maximum(deg_ref[:, 0:1], 1.0)
    m0 = a0_ref[...] * inv
    m1 = a1_ref[...] * inv
    h = (jnp.dot(m0, wl_ref[:128, :], preferred_element_type=jnp.float32)
         + jnp.dot(m1, wl_ref[128:, :], preferred_element_type=jnp.float32)
         + r_ref[...])
    mu = jnp.mean(h, axis=-1, keepdims=True)
    dlt = h - mu
    var = jnp.mean(dlt * dlt, axis=-1, keepdims=True)
    hn = dlt * lax.rsqrt(var + 1e-5) * g_ref[...] + b_ref[...]
    ge = 0.5 * hn * (1.0 + lax.erf(hn * 0.7071067811865476))
    o_ref[...] = ge + x_ref[...]


def _tc_finish(agg, deg, r, x, W_l, gamma, beta):
    nb = NP // BN
    return pl.pallas_call(
        _tc_body,
        grid=(N // BN + 1,),
        in_specs=[
            pl.BlockSpec((BN, DH), lambda i: (i, 0)),
            pl.BlockSpec((BN, DH), lambda i: (nb + i, 0)),
            pl.BlockSpec((BN, DG), lambda i: (i, 0)),
            pl.BlockSpec((BN, D), lambda i: (i, 0)),
            pl.BlockSpec((BN, D), lambda i: (i, 0)),
            pl.BlockSpec((D, D), lambda i: (0, 0)),
            pl.BlockSpec((1, D), lambda i: (0, 0)),
            pl.BlockSpec((1, D), lambda i: (0, 0)),
        ],
        out_specs=pl.BlockSpec((BN, D), lambda i: (i, 0)),
        out_shape=jax.ShapeDtypeStruct((N, D), jnp.float32),
    )(agg, agg, deg, r, x, W_l, gamma, beta)


def kernel(x, edge_index, W_l, b_l, W_r, gamma, beta):
    x2 = x.reshape(2 * N, DH)
    src = edge_index[0].astype(jnp.int32)
    dst = edge_index[1].astype(jnp.int32)
    ones8 = jnp.ones((CH, DG), jnp.float32)
    r = _tc_xwr(x, W_r, b_l.reshape(1, D))
    agg, deg = _sc_aggregate(x2, src, dst, ones8)
    return _tc_finish(agg, deg, r, x, W_l,
                      gamma.reshape(1, D), beta.reshape(1, D))


# speedup vs baseline: 1.0589x; 1.0090x over previous
import jax
import jax.numpy as jnp
from jax import lax
from jax.experimental import pallas as pl
from jax.experimental.pallas import tpu as pltpu
from jax.experimental.pallas import tpu_sc as plsc

N = 10000
E = 160000
D = 256
DH = 128
DG = 8
NSUB = 16
EPT = E // NSUB
CH = 80
NCH = EPT // CH
RPT = 632
NP = NSUB * RPT
BN = RPT

NBUF = 3
NIDX = 6


def _sc_agg_body(x_hbm, src_hbm, dst_hbm, ones_hbm, agg_hbm, deg_hbm,
                 agg_sh, deg_sh, ones_v, *rest):
    bufs = rest[0:NBUF]
    rvs = rest[NBUF:NBUF + NIDX]
    svs = rest[NBUF + NIDX:NBUF + 2 * NIDX]
    dvs = rest[NBUF + 2 * NIDX:NBUF + 3 * NIDX]
    semg = rest[NBUF + 3 * NIDX:2 * NBUF + 3 * NIDX]
    sems = rest[2 * NBUF + 3 * NIDX:3 * NBUF + 3 * NIDX]
    semd = rest[3 * NBUF + 3 * NIDX:4 * NBUF + 3 * NIDX]
    semi = rest[4 * NBUF + 3 * NIDX:4 * NBUF + 4 * NIDX]
    c = lax.axis_index("c")
    s = lax.axis_index("s")

    pltpu.sync_copy(ones_hbm, ones_v)

    def zfill(i, carry):
        for j in range(DH // 16):
            bufs[0][i, pl.ds(j * 16, 16)] = jnp.zeros((16,), jnp.float32)
        return carry

    lax.fori_loop(0, CH, zfill, 0)
    nz = RPT // CH
    for k in range(nz):
        pltpu.async_copy(bufs[0], agg_sh.at[pl.ds(s * RPT + k * CH, CH)],
                         semg[0])
    rz = RPT - nz * CH
    pltpu.async_copy(bufs[0].at[pl.ds(0, rz)],
                     agg_sh.at[pl.ds(s * RPT + nz * CH, rz)], semg[0])
    for k in range(8):
        pltpu.async_copy(bufs[0].at[pl.ds(0, RPT // 8), pl.ds(0, DG)],
                         deg_sh.at[pl.ds(s * RPT + k * (RPT // 8), RPT // 8)],
                         semg[1])
    for k in range(nz):
        pltpu.make_async_copy(bufs[0], agg_sh.at[pl.ds(0, CH)],
                              semg[0]).wait()
    pltpu.make_async_copy(bufs[0].at[pl.ds(0, rz)],
                          agg_sh.at[pl.ds(0, rz)], semg[0]).wait()
    for k in range(8):
        pltpu.make_async_copy(bufs[0].at[pl.ds(0, RPT // 8), pl.ds(0, DG)],
                              deg_sh.at[pl.ds(0, RPT // 8)], semg[1]).wait()
    plsc.subcore_barrier()

    ebase = s * EPT

    def prep(q, k):
        pltpu.async_copy(src_hbm.at[pl.ds(ebase + k * CH, CH)],
                         rvs[q], semi[q])
        pltpu.async_copy(dst_hbm.at[pl.ds(ebase + k * CH, CH)],
                         dvs[q], semi[q])

    def idxwait(q):
        pltpu.make_async_copy(src_hbm.at[pl.ds(0, CH)], rvs[q],
                              semi[q]).wait()
        pltpu.make_async_copy(dst_hbm.at[pl.ds(0, CH)], dvs[q],
                              semi[q]).wait()
        for j in range(CH // 16):
            v = rvs[q][pl.ds(j * 16, 16)]
            svs[q][pl.ds(j * 16, 16)] = v + v + c

    def gstart(b, q):
        pltpu.async_copy(x_hbm.at[svs[q]], bufs[b], semg[b])

    def gwait(b, q):
        pltpu.make_async_copy(x_hbm.at[svs[q]], bufs[b], semg[b]).wait()

    def sstart(b, q):
        pltpu.async_copy(bufs[b], agg_sh.at[dvs[q]], sems[b], add=True)
        pltpu.async_copy(ones_v, deg_sh.at[dvs[q]], semd[b], add=True)

    def swait(b, q):
        pltpu.make_async_copy(bufs[b], agg_sh.at[dvs[q]], sems[b]).wait()
        pltpu.make_async_copy(ones_v, deg_sh.at[dvs[q]], semd[b]).wait()

    def step(k, kk):
        b, q = kk % NBUF, kk % NIDX
        swait((kk - 1) % NBUF, (kk - 1) % NIDX)
        prep((kk + 5) % NIDX, k + 5)
        q2, b2 = (kk + 2) % NIDX, (kk + 2) % NBUF
        idxwait(q2)
        gstart(b2, q2)
        gwait(b, q)
        sstart(b, q)

    for k in range(5):
        prep(k % NIDX, k)
    idxwait(0)
    gstart(0, 0)
    idxwait(1)
    gstart(1, 1)
    prep(5, 5)
    idxwait(2)
    gstart(2, 2)
    gwait(0, 0)
    sstart(0, 0)

    NU = (NCH - 11) // 6

    def six(g, carry):
        k0 = 6 * g + 1
        for j in range(6):
            step(k0 + j, 1 + j)
        return carry

    lax.fori_loop(0, NU, six, 0)

    for k in range(6 * NU + 1, NCH - 5):
        step(k, k)

    for k in range(NCH - 5, NCH):
        b, q = k % NBUF, k % NIDX
        swait((k - 1) % NBUF, (k - 1) % NIDX)
        if k + 2 <= NCH - 1:
            q2, b2 = (k + 2) % NIDX, (k + 2) % NBUF
            idxwait(q2)
            gstart(b2, q2)
        gwait(b, q)
        sstart(b, q)
    swait((NCH - 1) % NBUF, (NCH - 1) % NIDX)
    plsc.subcore_barrier()

    pltpu.async_copy(deg_sh.at[pl.ds(s * RPT, RPT)],
                     deg_hbm.at[pl.ds(c * NP + s * RPT, RPT)], semg[1])
    pltpu.async_copy(agg_sh.at[pl.ds(s * RPT, RPT)],
                     agg_hbm.at[pl.ds(c * NP + s * RPT, RPT)], semg[0])
    pltpu.make_async_copy(deg_sh.at[pl.ds(s * RPT, RPT)],
                          deg_hbm.at[pl.ds(0, RPT)], semg[1]).wait()
    pltpu.make_async_copy(agg_sh.at[pl.ds(s * RPT, RPT)],
                          agg_hbm.at[pl.ds(0, RPT)], semg[0]).wait()


def _sc_aggregate(x2, src, dst, ones8):
    mesh = plsc.VectorSubcoreMesh(core_axis_name="c", subcore_axis_name="s")
    scratch = (
        [pltpu.VMEM_SHARED((NP, DH), jnp.float32),
         pltpu.VMEM_SHARED((NP, DG), jnp.float32),
         pltpu.VMEM((CH, DG), jnp.float32)]
        + [pltpu.VMEM((CH, DH), jnp.float32)] * NBUF
        + [pltpu.VMEM((CH,), jnp.int32)] * NIDX
        + [pltpu.VMEM((CH,), jnp.int32)] * NIDX
        + [pltpu.VMEM((CH,), jnp.int32)] * NIDX
        + [pltpu.SemaphoreType.DMA] * (3 * NBUF + NIDX)
    )
    return pl.kernel(
        _sc_agg_body,
        out_type=(jax.ShapeDtypeStruct((2 * NP, DH), jnp.float32),
                  jax.ShapeDtypeStruct((2 * NP, DG), jnp.float32)),
        mesh=mesh,
        scratch_types=scratch,
        compiler_params=pltpu.CompilerParams(use_tc_tiling_on_sc=False),
    )(x2, src, dst, ones8)


def _tc_xwr_body(x_ref, wr_ref, bl_ref, o_ref):
    o_ref[...] = (jnp.dot(x_ref[...], wr_ref[...],
                          preferred_element_type=jnp.float32)
                  + bl_ref[...]).astype(jnp.bfloat16)


def _tc_xwr(x, W_r, b_l):
    return pl.pallas_call(
        _tc_xwr_body,
        grid=(N // BN + 1,),
        in_specs=[
            pl.BlockSpec((BN, D), lambda i: (i, 0)),
            pl.BlockSpec((D, D), lambda i: (0, 0)),
            pl.BlockSpec((1, D), lambda i: (0, 0)),
        ],
        out_specs=pl.BlockSpec((BN, D), lambda i: (i, 0)),
        out_shape=jax.ShapeDtypeStruct((N, D), jnp.bfloat16),
    )(x, W_r, b_l)


def _tc_body(a0_ref, a1_ref, deg_ref, r_ref, x_ref, wl_ref, g_ref, b_ref,
             o_ref):
    inv = 1.0 / jnp.maximum(deg_ref[:, 0:1], 1.0)
    m0 = (a0_ref[...] * inv).astype(jnp.bfloat16)
    m1 = (a1_ref[...] * inv).astype(jnp.bfloat16)
    h = (jnp.dot(m0, wl_ref[:128, :], preferred_element_type=jnp.float32)
         + jnp.dot(m1, wl_ref[128:, :], preferred_element_type=jnp.float32)
         + r_ref[...].astype(jnp.float32))
    mu = jnp.mean(h, axis=-1, keepdims=True)
    dlt = h - mu
    var = jnp.mean(dlt * dlt, axis=-1, keepdims=True)
    hn = dlt * lax.rsqrt(var + 1e-5) * g_ref[...] + b_ref[...]
    ge = 0.5 * hn * (1.0 + lax.erf(hn * 0.7071067811865476))
    o_ref[...] = ge + x_ref[...]


def _tc_finish(agg, deg, r, x, W_l, gamma, beta):
    nb = NP // BN
    return pl.pallas_call(
        _tc_body,
        grid=(N // BN + 1,),
        in_specs=[
            pl.BlockSpec((BN, DH), lambda i: (i, 0)),
            pl.BlockSpec((BN, DH), lambda i: (nb + i, 0)),
            pl.BlockSpec((BN, DG), lambda i: (i, 0)),
            pl.BlockSpec((BN, D), lambda i: (i, 0)),
            pl.BlockSpec((BN, D), lambda i: (i, 0)),
            pl.BlockSpec((D, D), lambda i: (0, 0)),
            pl.BlockSpec((1, D), lambda i: (0, 0)),
            pl.BlockSpec((1, D), lambda i: (0, 0)),
        ],
        out_specs=pl.BlockSpec((BN, D), lambda i: (i, 0)),
        out_shape=jax.ShapeDtypeStruct((N, D), jnp.float32),
    )(agg, agg, deg, r, x, W_l, gamma, beta)


def kernel(x, edge_index, W_l, b_l, W_r, gamma, beta):
    x2 = x.reshape(2 * N, DH)
    src = edge_index[0].astype(jnp.int32)
    dst = edge_index[1].astype(jnp.int32)
    ones8 = jnp.ones((CH, DG), jnp.float32)
    r = _tc_xwr(x, W_r, b_l.reshape(1, D))
    agg, deg = _sc_aggregate(x2, src, dst, ones8)
    return _tc_finish(agg, deg, r, x, W_l.astype(jnp.bfloat16),
                      gamma.reshape(1, D), beta.reshape(1, D))


# speedup vs baseline: 1.0658x; 1.0065x over previous
import jax
import jax.numpy as jnp
from jax import lax
from jax.experimental import pallas as pl
from jax.experimental.pallas import tpu as pltpu
from jax.experimental.pallas import tpu_sc as plsc

N = 10000
E = 160000
D = 256
DH = 128
DG = 8
NSUB = 16
EPT = E // NSUB
CH = 80
NCH = EPT // CH
RPT = 632
NP = NSUB * RPT
BN = RPT

NBUF = 3
NIDX = 6


def _sc_agg_body(x_hbm, src_hbm, dst_hbm, ones_hbm, agg_hbm, deg_hbm,
                 agg_sh, deg_sh, ones_v, *rest):
    bufs = rest[0:NBUF]
    rvs = rest[NBUF:NBUF + NIDX]
    svs = rest[NBUF + NIDX:NBUF + 2 * NIDX]
    dvs = rest[NBUF + 2 * NIDX:NBUF + 3 * NIDX]
    semg = rest[NBUF + 3 * NIDX:2 * NBUF + 3 * NIDX]
    sems = rest[2 * NBUF + 3 * NIDX:3 * NBUF + 3 * NIDX]
    semd = rest[3 * NBUF + 3 * NIDX:4 * NBUF + 3 * NIDX]
    semi = rest[4 * NBUF + 3 * NIDX:4 * NBUF + 4 * NIDX]
    c = lax.axis_index("c")
    s = lax.axis_index("s")

    pltpu.sync_copy(ones_hbm, ones_v)

    def zfill(i, carry):
        for j in range(DH // 16):
            bufs[0][i, pl.ds(j * 16, 16)] = jnp.zeros((16,), jnp.float32)
        return carry

    lax.fori_loop(0, CH, zfill, 0)
    nz = RPT // CH
    rz = RPT - nz * CH
    for k in range(nz):
        pltpu.async_copy(bufs[0], agg_sh.at[pl.ds(s * RPT + k * CH, CH)],
                         sems[0])
    pltpu.async_copy(bufs[0].at[pl.ds(0, rz)],
                     agg_sh.at[pl.ds(s * RPT + nz * CH, rz)], sems[0])
    for k in range(8):
        pltpu.async_copy(bufs[0].at[pl.ds(0, RPT // 8), pl.ds(0, DG)],
                         deg_sh.at[pl.ds(s * RPT + k * (RPT // 8), RPT // 8)],
                         sems[1])

    ebase = s * EPT

    def prep(q, k):
        pltpu.async_copy(src_hbm.at[pl.ds(ebase + k * CH, CH)],
                         rvs[q], semi[q])
        pltpu.async_copy(dst_hbm.at[pl.ds(ebase + k * CH, CH)],
                         dvs[q], semi[q])

    def idxwait(q):
        pltpu.make_async_copy(src_hbm.at[pl.ds(0, CH)], rvs[q],
                              semi[q]).wait()
        pltpu.make_async_copy(dst_hbm.at[pl.ds(0, CH)], dvs[q],
                              semi[q]).wait()
        for j in range(CH // 16):
            v = rvs[q][pl.ds(j * 16, 16)]
            svs[q][pl.ds(j * 16, 16)] = v + v + c

    def gstart(b, q):
        pltpu.async_copy(x_hbm.at[svs[q]], bufs[b], semg[b])

    def gwait(b, q):
        pltpu.make_async_copy(x_hbm.at[svs[q]], bufs[b], semg[b]).wait()

    def sstart(b, q):
        pltpu.async_copy(bufs[b], agg_sh.at[dvs[q]], sems[b], add=True)
        pltpu.async_copy(ones_v, deg_sh.at[dvs[q]], semd[b], add=True)

    def swait(b, q):
        pltpu.make_async_copy(bufs[b], agg_sh.at[dvs[q]], sems[b]).wait()
        pltpu.make_async_copy(ones_v, deg_sh.at[dvs[q]], semd[b]).wait()

    def step(k, kk):
        b, q = kk % NBUF, kk % NIDX
        swait((kk - 1) % NBUF, (kk - 1) % NIDX)
        prep((kk + 5) % NIDX, k + 5)
        q2, b2 = (kk + 2) % NIDX, (kk + 2) % NBUF
        idxwait(q2)
        gstart(b2, q2)
        gwait(b, q)
        sstart(b, q)

    for k in range(5):
        prep(k % NIDX, k)
    idxwait(1)
    gstart(1, 1)
    prep(5, 5)
    idxwait(2)
    gstart(2, 2)
    for k in range(nz):
        pltpu.make_async_copy(bufs[0], agg_sh.at[pl.ds(0, CH)],
                              sems[0]).wait()
    pltpu.make_async_copy(bufs[0].at[pl.ds(0, rz)],
                          agg_sh.at[pl.ds(0, rz)], sems[0]).wait()
    for k in range(8):
        pltpu.make_async_copy(bufs[0].at[pl.ds(0, RPT // 8), pl.ds(0, DG)],
                              deg_sh.at[pl.ds(0, RPT // 8)], sems[1]).wait()
    idxwait(0)
    gstart(0, 0)
    plsc.subcore_barrier()
    gwait(0, 0)
    sstart(0, 0)

    NU = (NCH - 11) // 6

    def six(g, carry):
        k0 = 6 * g + 1
        for j in range(6):
            step(k0 + j, 1 + j)
        return carry

    lax.fori_loop(0, NU, six, 0)

    for k in range(6 * NU + 1, NCH - 5):
        step(k, k)

    for k in range(NCH - 5, NCH):
        b, q = k % NBUF, k % NIDX
        swait((k - 1) % NBUF, (k - 1) % NIDX)
        if k + 2 <= NCH - 1:
            q2, b2 = (k + 2) % NIDX, (k + 2) % NBUF
            idxwait(q2)
            gstart(b2, q2)
        gwait(b, q)
        sstart(b, q)
    swait((NCH - 1) % NBUF, (NCH - 1) % NIDX)
    plsc.subcore_barrier()

    pltpu.async_copy(deg_sh.at[pl.ds(s * RPT, RPT)],
                     deg_hbm.at[pl.ds(c * NP + s * RPT, RPT)], semg[1])
    pltpu.async_copy(agg_sh.at[pl.ds(s * RPT, RPT)],
                     agg_hbm.at[pl.ds(c * NP + s * RPT, RPT)], semg[0])
    pltpu.make_async_copy(deg_sh.at[pl.ds(s * RPT, RPT)],
                          deg_hbm.at[pl.ds(0, RPT)], semg[1]).wait()
    pltpu.make_async_copy(agg_sh.at[pl.ds(s * RPT, RPT)],
                          agg_hbm.at[pl.ds(0, RPT)], semg[0]).wait()


def _sc_aggregate(x2, src, dst, ones8):
    mesh = plsc.VectorSubcoreMesh(core_axis_name="c", subcore_axis_name="s")
    scratch = (
        [pltpu.VMEM_SHARED((NP, DH), jnp.float32),
         pltpu.VMEM_SHARED((NP, DG), jnp.float32),
         pltpu.VMEM((CH, DG), jnp.float32)]
        + [pltpu.VMEM((CH, DH), jnp.float32)] * NBUF
        + [pltpu.VMEM((CH,), jnp.int32)] * NIDX
        + [pltpu.VMEM((CH,), jnp.int32)] * NIDX
        + [pltpu.VMEM((CH,), jnp.int32)] * NIDX
        + [pltpu.SemaphoreType.DMA] * (3 * NBUF + NIDX)
    )
    return pl.kernel(
        _sc_agg_body,
        out_type=(jax.ShapeDtypeStruct((2 * NP, DH), jnp.float32),
                  jax.ShapeDtypeStruct((2 * NP, DG), jnp.float32)),
        mesh=mesh,
        scratch_types=scratch,
        compiler_params=pltpu.CompilerParams(use_tc_tiling_on_sc=False),
    )(x2, src, dst, ones8)


def _tc_xwr_body(x_ref, wr_ref, bl_ref, o_ref):
    o_ref[...] = (jnp.dot(x_ref[...], wr_ref[...],
                          preferred_element_type=jnp.float32)
                  + bl_ref[...]).astype(jnp.bfloat16)


def _tc_xwr(x, W_r, b_l):
    return pl.pallas_call(
        _tc_xwr_body,
        grid=(N // BN + 1,),
        in_specs=[
            pl.BlockSpec((BN, D), lambda i: (i, 0)),
            pl.BlockSpec((D, D), lambda i: (0, 0)),
            pl.BlockSpec((1, D), lambda i: (0, 0)),
        ],
        out_specs=pl.BlockSpec((BN, D), lambda i: (i, 0)),
        out_shape=jax.ShapeDtypeStruct((N, D), jnp.bfloat16),
    )(x, W_r, b_l)


def _tc_body(a0_ref, a1_ref, deg_ref, r_ref, x_ref, wl_ref, g_ref, b_ref,
             o_ref):
    inv = 1.0 / jnp.maximum(deg_ref[:, 0:1], 1.0)
    m0 = (a0_ref[...] * inv).astype(jnp.bfloat16)
    m1 = (a1_ref[...] * inv).astype(jnp.bfloat16)
    h = (jnp.dot(m0, wl_ref[:128, :], preferred_element_type=jnp.float32)
         + jnp.dot(m1, wl_ref[128:, :], preferred_element_type=jnp.float32)
         + r_ref[...].astype(jnp.float32))
    mu = jnp.mean(h, axis=-1, keepdims=True)
    dlt = h - mu
    var = jnp.mean(dlt * dlt, axis=-1, keepdims=True)
    hn = dlt * lax.rsqrt(var + 1e-5) * g_ref[...] + b_ref[...]
    ge = 0.5 * hn * (1.0 + lax.erf(hn * 0.7071067811865476))
    o_ref[...] = ge + x_ref[...]


def _tc_finish(agg, deg, r, x, W_l, gamma, beta):
    nb = NP // BN
    return pl.pallas_call(
        _tc_body,
        grid=(N // BN + 1,),
        in_specs=[
            pl.BlockSpec((BN, DH), lambda i: (i, 0)),
            pl.BlockSpec((BN, DH), lambda i: (nb + i, 0)),
            pl.BlockSpec((BN, DG), lambda i: (i, 0)),
            pl.BlockSpec((BN, D), lambda i: (i, 0)),
            pl.BlockSpec((BN, D), lambda i: (i, 0)),
            pl.BlockSpec((D, D), lambda i: (0, 0)),
            pl.BlockSpec((1, D), lambda i: (0, 0)),
            pl.BlockSpec((1, D), lambda i: (0, 0)),
        ],
        out_specs=pl.BlockSpec((BN, D), lambda i: (i, 0)),
        out_shape=jax.ShapeDtypeStruct((N, D), jnp.float32),
    )(agg, agg, deg, r, x, W_l, gamma, beta)


def kernel(x, edge_index, W_l, b_l, W_r, gamma, beta):
    x2 = x.reshape(2 * N, DH)
    src = edge_index[0].astype(jnp.int32)
    dst = edge_index[1].astype(jnp.int32)
    ones8 = jnp.ones((CH, DG), jnp.float32)
    r = _tc_xwr(x, W_r, b_l.reshape(1, D))
    agg, deg = _sc_aggregate(x2, src, dst, ones8)
    return _tc_finish(agg, deg, r, x, W_l.astype(jnp.bfloat16),
                      gamma.reshape(1, D), beta.reshape(1, D))


# speedup vs baseline: 1.0981x; 1.0303x over previous
import jax
import jax.numpy as jnp
from jax import lax
from jax.experimental import pallas as pl
from jax.experimental.pallas import tpu as pltpu
from jax.experimental.pallas import tpu_sc as plsc

N = 10000
E = 160000
D = 256
DH = 128
DG = 8
NSUB = 16
EPT = E // NSUB
CH = 80
NCH = EPT // CH
RPT = 632
NP = NSUB * RPT
BN = 2 * RPT

NBUF = 3
NIDX = 6


def _sc_agg_body(x_hbm, src_hbm, dst_hbm, ones_hbm, agg_hbm, deg_hbm,
                 agg_sh, deg_sh, ones_v, *rest):
    bufs = rest[0:NBUF]
    rvs = rest[NBUF:NBUF + NIDX]
    svs = rest[NBUF + NIDX:NBUF + 2 * NIDX]
    dvs = rest[NBUF + 2 * NIDX:NBUF + 3 * NIDX]
    semg = rest[NBUF + 3 * NIDX:2 * NBUF + 3 * NIDX]
    sems = rest[2 * NBUF + 3 * NIDX:3 * NBUF + 3 * NIDX]
    semd = rest[3 * NBUF + 3 * NIDX:4 * NBUF + 3 * NIDX]
    semi = rest[4 * NBUF + 3 * NIDX:4 * NBUF + 4 * NIDX]
    c = lax.axis_index("c")
    s = lax.axis_index("s")

    pltpu.sync_copy(ones_hbm, ones_v)

    def zfill(i, carry):
        for j in range(DH // 16):
            bufs[0][i, pl.ds(j * 16, 16)] = jnp.zeros((16,), jnp.float32)
        return carry

    lax.fori_loop(0, CH, zfill, 0)
    nz = RPT // CH
    rz = RPT - nz * CH
    for k in range(nz):
        pltpu.async_copy(bufs[0], agg_sh.at[pl.ds(s * RPT + k * CH, CH)],
                         sems[0])
    pltpu.async_copy(bufs[0].at[pl.ds(0, rz)],
                     agg_sh.at[pl.ds(s * RPT + nz * CH, rz)], sems[0])
    for k in range(8):
        pltpu.async_copy(bufs[0].at[pl.ds(0, RPT // 8), pl.ds(0, DG)],
                         deg_sh.at[pl.ds(s * RPT + k * (RPT // 8), RPT // 8)],
                         sems[1])

    ebase = s * EPT

    def prep(q, k):
        pltpu.async_copy(src_hbm.at[pl.ds(ebase + k * CH, CH)],
                         rvs[q], semi[q])
        pltpu.async_copy(dst_hbm.at[pl.ds(ebase + k * CH, CH)],
                         dvs[q], semi[q])

    def idxwait(q):
        pltpu.make_async_copy(src_hbm.at[pl.ds(0, CH)], rvs[q],
                              semi[q]).wait()
        pltpu.make_async_copy(dst_hbm.at[pl.ds(0, CH)], dvs[q],
                              semi[q]).wait()
        for j in range(CH // 16):
            v = rvs[q][pl.ds(j * 16, 16)]
            svs[q][pl.ds(j * 16, 16)] = v + v + c

    def gstart(b, q):
        pltpu.async_copy(x_hbm.at[svs[q]], bufs[b], semg[b])

    def gwait(b, q):
        pltpu.make_async_copy(x_hbm.at[svs[q]], bufs[b], semg[b]).wait()

    def sstart(b, q):
        pltpu.async_copy(bufs[b], agg_sh.at[dvs[q]], sems[b], add=True)
        pltpu.async_copy(ones_v, deg_sh.at[dvs[q]], semd[b], add=True)

    def swait(b, q):
        pltpu.make_async_copy(bufs[b], agg_sh.at[dvs[q]], sems[b]).wait()
        pltpu.make_async_copy(ones_v, deg_sh.at[dvs[q]], semd[b]).wait()

    def step(k, kk):
        b, q = kk % NBUF, kk % NIDX
        swait((kk - 1) % NBUF, (kk - 1) % NIDX)
        prep((kk + 5) % NIDX, k + 5)
        q2, b2 = (kk + 2) % NIDX, (kk + 2) % NBUF
        idxwait(q2)
        gstart(b2, q2)
        gwait(b, q)
        sstart(b, q)

    for k in range(5):
        prep(k % NIDX, k)
    idxwait(1)
    gstart(1, 1)
    prep(5, 5)
    idxwait(2)
    gstart(2, 2)
    for k in range(nz):
        pltpu.make_async_copy(bufs[0], agg_sh.at[pl.ds(0, CH)],
                              sems[0]).wait()
    pltpu.make_async_copy(bufs[0].at[pl.ds(0, rz)],
                          agg_sh.at[pl.ds(0, rz)], sems[0]).wait()
    for k in range(8):
        pltpu.make_async_copy(bufs[0].at[pl.ds(0, RPT // 8), pl.ds(0, DG)],
                              deg_sh.at[pl.ds(0, RPT // 8)], sems[1]).wait()
    idxwait(0)
    gstart(0, 0)
    plsc.subcore_barrier()
    gwait(0, 0)
    sstart(0, 0)

    NU = (NCH - 11) // 6

    def six(g, carry):
        k0 = 6 * g + 1
        for j in range(6):
            step(k0 + j, 1 + j)
        return carry

    lax.fori_loop(0, NU, six, 0)

    for k in range(6 * NU + 1, NCH - 5):
        step(k, k)

    for k in range(NCH - 5, NCH):
        b, q = k % NBUF, k % NIDX
        swait((k - 1) % NBUF, (k - 1) % NIDX)
        if k + 2 <= NCH - 1:
            q2, b2 = (k + 2) % NIDX, (k + 2) % NBUF
            idxwait(q2)
            gstart(b2, q2)
        gwait(b, q)
        sstart(b, q)
    swait((NCH - 1) % NBUF, (NCH - 1) % NIDX)
    plsc.subcore_barrier()

    pltpu.async_copy(deg_sh.at[pl.ds(s * RPT, RPT)],
                     deg_hbm.at[pl.ds(c * NP + s * RPT, RPT)], semg[1])
    pltpu.async_copy(agg_sh.at[pl.ds(s * RPT, RPT)],
                     agg_hbm.at[pl.ds(c * NP + s * RPT, RPT)], semg[0])
    pltpu.make_async_copy(deg_sh.at[pl.ds(s * RPT, RPT)],
                          deg_hbm.at[pl.ds(0, RPT)], semg[1]).wait()
    pltpu.make_async_copy(agg_sh.at[pl.ds(s * RPT, RPT)],
                          agg_hbm.at[pl.ds(0, RPT)], semg[0]).wait()


def _sc_aggregate(x2, src, dst, ones8):
    mesh = plsc.VectorSubcoreMesh(core_axis_name="c", subcore_axis_name="s")
    scratch = (
        [pltpu.VMEM_SHARED((NP, DH), jnp.float32),
         pltpu.VMEM_SHARED((NP, DG), jnp.float32),
         pltpu.VMEM((CH, DG), jnp.float32)]
        + [pltpu.VMEM((CH, DH), jnp.float32)] * NBUF
        + [pltpu.VMEM((CH,), jnp.int32)] * NIDX
        + [pltpu.VMEM((CH,), jnp.int32)] * NIDX
        + [pltpu.VMEM((CH,), jnp.int32)] * NIDX
        + [pltpu.SemaphoreType.DMA] * (3 * NBUF + NIDX)
    )
    return pl.kernel(
        _sc_agg_body,
        out_type=(jax.ShapeDtypeStruct((2 * NP, DH), jnp.float32),
                  jax.ShapeDtypeStruct((2 * NP, DG), jnp.float32)),
        mesh=mesh,
        scratch_types=scratch,
        compiler_params=pltpu.CompilerParams(use_tc_tiling_on_sc=False),
    )(x2, src, dst, ones8)


def _tc_xwr_body(x_ref, wr_ref, bl_ref, o_ref):
    o_ref[...] = (jnp.dot(x_ref[...], wr_ref[...],
                          preferred_element_type=jnp.float32)
                  + bl_ref[...]).astype(jnp.bfloat16)


def _tc_xwr(x, W_r, b_l):
    return pl.pallas_call(
        _tc_xwr_body,
        grid=(N // BN + 1,),
        in_specs=[
            pl.BlockSpec((BN, D), lambda i: (i, 0)),
            pl.BlockSpec((D, D), lambda i: (0, 0)),
            pl.BlockSpec((1, D), lambda i: (0, 0)),
        ],
        out_specs=pl.BlockSpec((BN, D), lambda i: (i, 0)),
        out_shape=jax.ShapeDtypeStruct((N, D), jnp.bfloat16),
    )(x, W_r, b_l)


def _tc_body(a0_ref, a1_ref, deg_ref, r_ref, x_ref, wl_ref, g_ref, b_ref,
             o_ref):
    inv = 1.0 / jnp.maximum(deg_ref[:, 0:1], 1.0)
    m0 = (a0_ref[...] * inv).astype(jnp.bfloat16)
    m1 = (a1_ref[...] * inv).astype(jnp.bfloat16)
    h = (jnp.dot(m0, wl_ref[:128, :], preferred_element_type=jnp.float32)
         + jnp.dot(m1, wl_ref[128:, :], preferred_element_type=jnp.float32)
         + r_ref[...].astype(jnp.float32))
    mu = jnp.mean(h, axis=-1, keepdims=True)
    dlt = h - mu
    var = jnp.mean(dlt * dlt, axis=-1, keepdims=True)
    hn = dlt * lax.rsqrt(var + 1e-5) * g_ref[...] + b_ref[...]
    ge = 0.5 * hn * (1.0 + lax.erf(hn * 0.7071067811865476))
    o_ref[...] = ge + x_ref[...]


def _tc_finish(agg, deg, r, x, W_l, gamma, beta):
    nb = NP // BN
    return pl.pallas_call(
        _tc_body,
        grid=(N // BN + 1,),
        in_specs=[
            pl.BlockSpec((BN, DH), lambda i: (i, 0)),
            pl.BlockSpec((BN, DH), lambda i: (nb + i, 0)),
            pl.BlockSpec((BN, DG), lambda i: (i, 0)),
            pl.BlockSpec((BN, D), lambda i: (i, 0)),
            pl.BlockSpec((BN, D), lambda i: (i, 0)),
            pl.BlockSpec((D, D), lambda i: (0, 0)),
            pl.BlockSpec((1, D), lambda i: (0, 0)),
            pl.BlockSpec((1, D), lambda i: (0, 0)),
        ],
        out_specs=pl.BlockSpec((BN, D), lambda i: (i, 0)),
        out_shape=jax.ShapeDtypeStruct((N, D), jnp.float32),
    )(agg, agg, deg, r, x, W_l, gamma, beta)


def kernel(x, edge_index, W_l, b_l, W_r, gamma, beta):
    x2 = x.reshape(2 * N, DH)
    src = edge_index[0].astype(jnp.int32)
    dst = edge_index[1].astype(jnp.int32)
    ones8 = jnp.ones((CH, DG), jnp.float32)
    r = _tc_xwr(x, W_r, b_l.reshape(1, D))
    agg, deg = _sc_aggregate(x2, src, dst, ones8)
    return _tc_finish(agg, deg, r, x, W_l.astype(jnp.bfloat16),
                      gamma.reshape(1, D), beta.reshape(1, D))


# speedup vs baseline: 1.1019x; 1.0034x over previous
import jax
import jax.numpy as jnp
from jax import lax
from jax.experimental import pallas as pl
from jax.experimental.pallas import tpu as pltpu
from jax.experimental.pallas import tpu_sc as plsc

N = 10000
E = 160000
D = 256
DH = 128
DG = 8
NSUB = 16
EPT = E // NSUB
CH = 80
NCH = EPT // CH
RPT = 632
NP = NSUB * RPT
BN = 4 * RPT

NBUF = 3
NIDX = 6


def _sc_agg_body(x_hbm, src_hbm, dst_hbm, ones_hbm, agg_hbm, deg_hbm,
                 agg_sh, deg_sh, ones_v, *rest):
    bufs = rest[0:NBUF]
    rvs = rest[NBUF:NBUF + NIDX]
    svs = rest[NBUF + NIDX:NBUF + 2 * NIDX]
    dvs = rest[NBUF + 2 * NIDX:NBUF + 3 * NIDX]
    semg = rest[NBUF + 3 * NIDX:2 * NBUF + 3 * NIDX]
    sems = rest[2 * NBUF + 3 * NIDX:3 * NBUF + 3 * NIDX]
    semd = rest[3 * NBUF + 3 * NIDX:4 * NBUF + 3 * NIDX]
    semi = rest[4 * NBUF + 3 * NIDX:4 * NBUF + 4 * NIDX]
    c = lax.axis_index("c")
    s = lax.axis_index("s")

    pltpu.sync_copy(ones_hbm, ones_v)

    def zfill(i, carry):
        for j in range(DH // 16):
            bufs[0][i, pl.ds(j * 16, 16)] = jnp.zeros((16,), jnp.float32)
        return carry

    lax.fori_loop(0, CH, zfill, 0)
    nz = RPT // CH
    rz = RPT - nz * CH
    for k in range(nz):
        pltpu.async_copy(bufs[0], agg_sh.at[pl.ds(s * RPT + k * CH, CH)],
                         sems[0])
    pltpu.async_copy(bufs[0].at[pl.ds(0, rz)],
                     agg_sh.at[pl.ds(s * RPT + nz * CH, rz)], sems[0])
    for k in range(8):
        pltpu.async_copy(bufs[0].at[pl.ds(0, RPT // 8), pl.ds(0, DG)],
                         deg_sh.at[pl.ds(s * RPT + k * (RPT // 8), RPT // 8)],
                         sems[1])

    ebase = s * EPT

    def prep(q, k):
        pltpu.async_copy(src_hbm.at[pl.ds(ebase + k * CH, CH)],
                         rvs[q], semi[q])
        pltpu.async_copy(dst_hbm.at[pl.ds(ebase + k * CH, CH)],
                         dvs[q], semi[q])

    def idxwait(q):
        pltpu.make_async_copy(src_hbm.at[pl.ds(0, CH)], rvs[q],
                              semi[q]).wait()
        pltpu.make_async_copy(dst_hbm.at[pl.ds(0, CH)], dvs[q],
                              semi[q]).wait()
        for j in range(CH // 16):
            v = rvs[q][pl.ds(j * 16, 16)]
            svs[q][pl.ds(j * 16, 16)] = v + v + c

    def gstart(b, q):
        pltpu.async_copy(x_hbm.at[svs[q]], bufs[b], semg[b])

    def gwait(b, q):
        pltpu.make_async_copy(x_hbm.at[svs[q]], bufs[b], semg[b]).wait()

    def sstart(b, q):
        pltpu.async_copy(bufs[b], agg_sh.at[dvs[q]], sems[b], add=True)
        pltpu.async_copy(ones_v, deg_sh.at[dvs[q]], semd[b], add=True)

    def swait(b, q):
        pltpu.make_async_copy(bufs[b], agg_sh.at[dvs[q]], sems[b]).wait()
        pltpu.make_async_copy(ones_v, deg_sh.at[dvs[q]], semd[b]).wait()

    def step(k, kk):
        b, q = kk % NBUF, kk % NIDX
        swait((kk - 1) % NBUF, (kk - 1) % NIDX)
        prep((kk + 5) % NIDX, k + 5)
        q2, b2 = (kk + 2) % NIDX, (kk + 2) % NBUF
        idxwait(q2)
        gstart(b2, q2)
        gwait(b, q)
        sstart(b, q)

    for k in range(5):
        prep(k % NIDX, k)
    idxwait(1)
    gstart(1, 1)
    prep(5, 5)
    idxwait(2)
    gstart(2, 2)
    for k in range(nz):
        pltpu.make_async_copy(bufs[0], agg_sh.at[pl.ds(0, CH)],
                              sems[0]).wait()
    pltpu.make_async_copy(bufs[0].at[pl.ds(0, rz)],
                          agg_sh.at[pl.ds(0, rz)], sems[0]).wait()
    for k in range(8):
        pltpu.make_async_copy(bufs[0].at[pl.ds(0, RPT // 8), pl.ds(0, DG)],
                              deg_sh.at[pl.ds(0, RPT // 8)], sems[1]).wait()
    idxwait(0)
    gstart(0, 0)
    plsc.subcore_barrier()
    gwait(0, 0)
    sstart(0, 0)

    NU = (NCH - 11) // 6

    def six(g, carry):
        k0 = 6 * g + 1
        for j in range(6):
            step(k0 + j, 1 + j)
        return carry

    lax.fori_loop(0, NU, six, 0)

    for k in range(6 * NU + 1, NCH - 5):
        step(k, k)

    for k in range(NCH - 5, NCH):
        b, q = k % NBUF, k % NIDX
        swait((k - 1) % NBUF, (k - 1) % NIDX)
        if k + 2 <= NCH - 1:
            q2, b2 = (k + 2) % NIDX, (k + 2) % NBUF
            idxwait(q2)
            gstart(b2, q2)
        gwait(b, q)
        sstart(b, q)
    swait((NCH - 1) % NBUF, (NCH - 1) % NIDX)
    plsc.subcore_barrier()

    pltpu.async_copy(deg_sh.at[pl.ds(s * RPT, RPT)],
                     deg_hbm.at[pl.ds(c * NP + s * RPT, RPT)], semg[1])
    pltpu.async_copy(agg_sh.at[pl.ds(s * RPT, RPT)],
                     agg_hbm.at[pl.ds(c * NP + s * RPT, RPT)], semg[0])
    pltpu.make_async_copy(deg_sh.at[pl.ds(s * RPT, RPT)],
                          deg_hbm.at[pl.ds(0, RPT)], semg[1]).wait()
    pltpu.make_async_copy(agg_sh.at[pl.ds(s * RPT, RPT)],
                          agg_hbm.at[pl.ds(0, RPT)], semg[0]).wait()


def _sc_aggregate(x2, src, dst, ones8):
    mesh = plsc.VectorSubcoreMesh(core_axis_name="c", subcore_axis_name="s")
    scratch = (
        [pltpu.VMEM_SHARED((NP, DH), jnp.float32),
         pltpu.VMEM_SHARED((NP, DG), jnp.float32),
         pltpu.VMEM((CH, DG), jnp.float32)]
        + [pltpu.VMEM((CH, DH), jnp.float32)] * NBUF
        + [pltpu.VMEM((CH,), jnp.int32)] * NIDX
        + [pltpu.VMEM((CH,), jnp.int32)] * NIDX
        + [pltpu.VMEM((CH,), jnp.int32)] * NIDX
        + [pltpu.SemaphoreType.DMA] * (3 * NBUF + NIDX)
    )
    return pl.kernel(
        _sc_agg_body,
        out_type=(jax.ShapeDtypeStruct((2 * NP, DH), jnp.float32),
                  jax.ShapeDtypeStruct((2 * NP, DG), jnp.float32)),
        mesh=mesh,
        scratch_types=scratch,
        compiler_params=pltpu.CompilerParams(use_tc_tiling_on_sc=False),
    )(x2, src, dst, ones8)


def _tc_xwr_body(x_ref, wr_ref, bl_ref, o_ref):
    o_ref[...] = (jnp.dot(x_ref[...], wr_ref[...],
                          preferred_element_type=jnp.float32)
                  + bl_ref[...]).astype(jnp.bfloat16)


def _tc_xwr(x, W_r, b_l):
    return pl.pallas_call(
        _tc_xwr_body,
        grid=(N // BN + 1,),
        in_specs=[
            pl.BlockSpec((BN, D), lambda i: (i, 0)),
            pl.BlockSpec((D, D), lambda i: (0, 0)),
            pl.BlockSpec((1, D), lambda i: (0, 0)),
        ],
        out_specs=pl.BlockSpec((BN, D), lambda i: (i, 0)),
        out_shape=jax.ShapeDtypeStruct((N, D), jnp.bfloat16),
    )(x, W_r, b_l)


def _tc_body(a0_ref, a1_ref, deg_ref, r_ref, x_ref, wl_ref, g_ref, b_ref,
             o_ref):
    inv = 1.0 / jnp.maximum(deg_ref[:, 0:1], 1.0)
    m0 = (a0_ref[...] * inv).astype(jnp.bfloat16)
    m1 = (a1_ref[...] * inv).astype(jnp.bfloat16)
    h = (jnp.dot(m0, wl_ref[:128, :], preferred_element_type=jnp.float32)
         + jnp.dot(m1, wl_ref[128:, :], preferred_element_type=jnp.float32)
         + r_ref[...].astype(jnp.float32))
    mu = jnp.mean(h, axis=-1, keepdims=True)
    dlt = h - mu
    var = jnp.mean(dlt * dlt, axis=-1, keepdims=True)
    hn = dlt * lax.rsqrt(var + 1e-5) * g_ref[...] + b_ref[...]
    ge = 0.5 * hn * (1.0 + lax.erf(hn * 0.7071067811865476))
    o_ref[...] = ge + x_ref[...]


def _tc_finish(agg, deg, r, x, W_l, gamma, beta):
    nb = NP // BN
    return pl.pallas_call(
        _tc_body,
        grid=(N // BN + 1,),
        in_specs=[
            pl.BlockSpec((BN, DH), lambda i: (i, 0)),
            pl.BlockSpec((BN, DH), lambda i: (nb + i, 0)),
            pl.BlockSpec((BN, DG), lambda i: (i, 0)),
            pl.BlockSpec((BN, D), lambda i: (i, 0)),
            pl.BlockSpec((BN, D), lambda i: (i, 0)),
            pl.BlockSpec((D, D), lambda i: (0, 0)),
            pl.BlockSpec((1, D), lambda i: (0, 0)),
            pl.BlockSpec((1, D), lambda i: (0, 0)),
        ],
        out_specs=pl.BlockSpec((BN, D), lambda i: (i, 0)),
        out_shape=jax.ShapeDtypeStruct((N, D), jnp.float32),
    )(agg, agg, deg, r, x, W_l, gamma, beta)


def kernel(x, edge_index, W_l, b_l, W_r, gamma, beta):
    x2 = x.reshape(2 * N, DH)
    src = edge_index[0].astype(jnp.int32)
    dst = edge_index[1].astype(jnp.int32)
    ones8 = jnp.ones((CH, DG), jnp.float32)
    r = _tc_xwr(x, W_r, b_l.reshape(1, D))
    agg, deg = _sc_aggregate(x2, src, dst, ones8)
    return _tc_finish(agg, deg, r, x, W_l.astype(jnp.bfloat16),
                      gamma.reshape(1, D), beta.reshape(1, D))
